# baseline (speedup 1.0000x reference)
import jax
import jax.numpy as jnp
from jax import lax
from jax.experimental import pallas as pl
from jax.experimental.pallas import tpu as pltpu
from jax.experimental.pallas import tpu_sc as plsc

NC, NS, LANES = 2, 16, 16
NW = NC * NS

B, F_, L_, V_, D_ = 1024, 26, 20, 1000, 64
N = B * F_ * L_
PERIOD = F_ * L_
SUB = 104
NSUB = PERIOD // SUB
SPC = 2
NCHUNK = B // SPC
NV16 = PERIOD // LANES


def _body(idx_hbm, table_hbm, off_hbm, out_hbm, idx_v, adj_v, rows_v, off_v,
          gsem0, gsem1, ssem0, ssem1, isem):
    wid = lax.axis_index("s") * NC + lax.axis_index("c")
    gsems = (gsem0, gsem1)
    ssems = (ssem0, ssem1)
    pltpu.sync_copy(off_hbm, off_v)
    n_mine = NCHUNK // NW

    def first_sample(g):
        return SPC * (g * NW + wid)

    def idx_load(g, slot):
        g = jnp.minimum(g, n_mine - 1)
        base = pl.multiple_of(first_sample(g) * PERIOD, SPC * PERIOD)
        return pltpu.make_async_copy(
            idx_hbm.at[pl.ds(base, SPC * PERIOD)], idx_v.at[slot], isem
        )

    def make_store(g, h):
        row0 = pl.multiple_of((first_sample(g) + h) * PERIOD, PERIOD)
        return pltpu.make_async_copy(
            rows_v.at[h],
            out_hbm.at[pl.ds(row0, PERIOD)],
            ssems[h],
        )

    idx_load(0, 0).start()
    idx_load(0, 0).wait()

    def chunk(g, first):
        slot = lax.rem(g, 2)
        idx_load(g + 1, 1 - slot).start()
        for h in range(SPC):
            hb = h * PERIOD
            starts = [k * LANES for k in range(NV16)] + [PERIOD - LANES]
            for s in starts:
                off = off_v[pl.ds(s, LANES)]
                adj_v[slot, pl.ds(hb + s, LANES)] = (
                    idx_v[slot, pl.ds(hb + s, LANES)] + off
                )
        for h in range(SPC):
            @pl.when(jnp.logical_not(first))
            def _():
                make_store(g, h).wait()
            for j in range(NSUB):
                pltpu.make_async_copy(
                    table_hbm.at[adj_v.at[slot, pl.ds(h * PERIOD + j * SUB, SUB)]],
                    rows_v.at[h, pl.ds(j * SUB, SUB)],
                    gsems[h],
                ).start()
        for h in range(SPC):
            for j in range(NSUB):
                pltpu.make_async_copy(
                    table_hbm.at[adj_v.at[slot, pl.ds(h * PERIOD + j * SUB, SUB)]],
                    rows_v.at[h, pl.ds(j * SUB, SUB)],
                    gsems[h],
                ).wait()
            make_store(g, h).start()
        idx_load(g + 1, 1 - slot).wait()
        return jnp.bool_(False)

    lax.fori_loop(0, n_mine, chunk, jnp.bool_(True))
    for h in range(SPC):
        make_store(0, h).wait()


@jax.jit
def kernel(indices, tables):
    flat_tables = tables.reshape(F_ * V_, D_)
    idx1d = indices.reshape(N)
    off_lut = (jnp.arange(PERIOD, dtype=jnp.int32) // L_) * V_
    mesh = plsc.VectorSubcoreMesh(
        core_axis_name="c", subcore_axis_name="s", num_cores=NC, num_subcores=NS
    )
    out = pl.kernel(
        _body,
        out_type=jax.ShapeDtypeStruct((N, D_), jnp.float32),
        mesh=mesh,
        scratch_types=[
            pltpu.VMEM((2, SPC * PERIOD), jnp.int32),
            pltpu.VMEM((2, SPC * PERIOD), jnp.int32),
            pltpu.VMEM((SPC, PERIOD, D_), jnp.float32),
            pltpu.VMEM((PERIOD,), jnp.int32),
            pltpu.SemaphoreType.DMA,
            pltpu.SemaphoreType.DMA,
            pltpu.SemaphoreType.DMA,
            pltpu.SemaphoreType.DMA,
            pltpu.SemaphoreType.DMA,
        ],
        compiler_params=pltpu.CompilerParams(use_tc_tiling_on_sc=False),
    )(idx1d, flat_tables, off_lut)
    return out.reshape(B, F_ * L_ * D_)
